# baseline (speedup 1.0000x reference)
import functools

import jax
import jax.numpy as jnp
from jax import lax
from jax.experimental import pallas as pl
from jax.experimental.pallas import tpu as pltpu
from jax.experimental.pallas import tpu_sc as plsc

D = 64
B = 4096
L = 16
NQ = D // L
MARGIN = 1.0

_info = plsc.get_sparse_core_info()
NC = _info.num_cores
NS = _info.num_subcores
NW = NC * NS
RPW = B // NW
CH = 8
NCHUNK = RPW // CH


def _lane_iota():
    return lax.iota(jnp.int32, L)


def _lanesum(v):
    iota = _lane_iota()
    for sh in (1, 2, 4, 8):
        v = v + v.at[iota ^ sh].get(mode="promise_in_bounds")
    return v


def _rsqrt(v):
    i = lax.bitcast_convert_type(v, jnp.int32)
    i = jnp.int32(0x5F3759DF) - (i >> 1)
    y = lax.bitcast_convert_type(i, jnp.float32)
    for _ in range(3):
        y = y * (1.5 - 0.5 * v * y * y)
    return y


def _bcast_lane(vec, j):
    return jnp.broadcast_to(vec[j], (L,))


def _row_score(M, H, T, R, rr, mh, mt, mr, l1f):
    def sel(A, f, lo):
        a = A[rr, pl.ds(lo, L)]
        b = A[rr, pl.ds(D + lo, L)]
        return a + f * (b - a)

    def dgrp(g, carry):
        accs = list(carry)
        hch = sel(H, mh, g * L)
        tch = sel(T, mt, g * L)
        for j in range(L):
            bh = _bcast_lane(hch, j)
            bt = _bcast_lane(tch, j)
            dd = g * L + j
            for q in range(NQ):
                m = M[rr, pl.ds(dd * D + q * L, L)]
                accs[q] = accs[q] + bh * m
                accs[NQ + q] = accs[NQ + q] + bt * m
        return tuple(accs)

    z = jnp.zeros((L,), jnp.float32)
    res = lax.fori_loop(0, D // L, dgrp, (z,) * (2 * NQ))
    ah, at = list(res[:NQ]), list(res[NQ:])

    def norm(a):
        ss = a[0] * a[0]
        for q in range(1, NQ):
            ss = ss + a[q] * a[q]
        sv = jnp.maximum(_lanesum(ss), jnp.float32(1e-12))
        y = _rsqrt(sv)
        return [aq * y for aq in a]

    ah = norm(ah)
    at = norm(at)
    sl1 = None
    sl2 = None
    for q in range(NQ):
        rq = sel(R, mr, q * L)
        dq = ah[q] + rq - at[q]
        aq = jnp.abs(dq)
        s2 = dq * dq
        sl1 = aq if sl1 is None else sl1 + aq
        sl2 = s2 if sl2 is None else sl2 + s2
    v1 = _lanesum(sl1)
    v2 = _lanesum(sl2)
    return l1f * v1 + (1.0 - l1f) * v2


def _sc_body(ph, pt, pr, nh, nt, nr, ent2, rel2, tr, l1h, out,
             bph, bpt, bnh, bnt, bpr, bnr, bprs, bnrs,
             oph, opt, opr, onh, ont, onr,
             eph, ept, enh, ent_, erp, ern, mp, mn, l1b, accb, sem):
    c = lax.axis_index("c")
    s = lax.axis_index("s")
    wid = s * NC + c
    base = wid * RPW

    pltpu.sync_copy(ph.at[pl.ds(base, RPW)], bph)
    pltpu.sync_copy(pt.at[pl.ds(base, RPW)], bpt)
    pltpu.sync_copy(nh.at[pl.ds(base, RPW)], bnh)
    pltpu.sync_copy(nt.at[pl.ds(base, RPW)], bnt)
    pltpu.sync_copy(pr.at[pl.ds(base, RPW)], bpr)
    pltpu.sync_copy(nr.at[pl.ds(base, RPW)], bnr)
    pltpu.sync_copy(l1h, l1b)
    l1f = l1b[...]

    def prep(k, _):
        s0 = k * L
        for buf, off in ((bph, oph), (bpt, opt), (bnh, onh), (bnt, ont)):
            v = buf[pl.ds(s0, L)]
            buf[pl.ds(s0, L)] = v >> 1
            off[pl.ds(s0, L)] = v & 1
        for buf, sh, off in ((bpr, bprs, opr), (bnr, bnrs, onr)):
            v = buf[pl.ds(s0, L)]
            sh[pl.ds(s0, L)] = v >> 1
            off[pl.ds(s0, L)] = v & 1
        return 0

    lax.fori_loop(0, RPW // L, prep, 0)

    def chunk(ci, acc):
        c0 = ci * CH
        cps = [
            pltpu.async_copy(ent2.at[bph.at[pl.ds(c0, CH)]], eph, sem),
            pltpu.async_copy(ent2.at[bpt.at[pl.ds(c0, CH)]], ept, sem),
            pltpu.async_copy(ent2.at[bnh.at[pl.ds(c0, CH)]], enh, sem),
            pltpu.async_copy(ent2.at[bnt.at[pl.ds(c0, CH)]], ent_, sem),
            pltpu.async_copy(rel2.at[bprs.at[pl.ds(c0, CH)]], erp, sem),
            pltpu.async_copy(rel2.at[bnrs.at[pl.ds(c0, CH)]], ern, sem),
            pltpu.async_copy(tr.at[bpr.at[pl.ds(c0, CH)]], mp, sem),
            pltpu.async_copy(tr.at[bnr.at[pl.ds(c0, CH)]], mn, sem),
        ]
        for cp in cps:
            cp.wait()

        def row(rr, acc2):
            g = c0 + rr
            wb = (g >> 4) << 4
            ln = jnp.broadcast_to(g & (L - 1), (L,))

            def mask_of(obuf):
                w = obuf[pl.ds(wb, L)]
                return w.at[ln].get(mode="promise_in_bounds").astype(jnp.float32)

            posv = _row_score(mp, eph, ept, erp, rr,
                              mask_of(oph), mask_of(opt), mask_of(opr), l1f)
            negv = _row_score(mn, enh, ent_, ern, rr,
                              mask_of(onh), mask_of(ont), mask_of(onr), l1f)
            return acc2 + jnp.maximum(posv - negv + MARGIN, 0.0)

        return lax.fori_loop(0, CH, row, acc)

    acc = lax.fori_loop(0, NCHUNK, chunk, jnp.zeros((L,), jnp.float32))
    accb[...] = acc
    pltpu.sync_copy(accb, out.at[wid])


_sc_call = functools.partial(
    pl.kernel,
    out_type=jax.ShapeDtypeStruct((NW, L), jnp.float32),
    mesh=plsc.VectorSubcoreMesh(core_axis_name="c", subcore_axis_name="s"),
    scratch_types=(
        [pltpu.VMEM((RPW,), jnp.int32) for _ in range(14)] +
        [pltpu.VMEM((CH, 2 * D), jnp.float32) for _ in range(6)] +
        [pltpu.VMEM((CH, D * D), jnp.float32) for _ in range(2)] +
        [pltpu.VMEM((L,), jnp.float32) for _ in range(2)] +
        [pltpu.SemaphoreType.DMA]
    ),
)(_sc_body)


def kernel(x, ent_emb, rel_emb, transfer, l1_flag):
    ph = x[:, 0]
    pt = x[:, 1]
    pr = x[:, 2]
    nh = x[:, 3]
    nt = x[:, 4]
    nr = x[:, 5]
    ent2 = ent_emb.reshape(ent_emb.shape[0] // 2, 2 * D)
    rel2 = rel_emb.reshape(rel_emb.shape[0] // 2, 2 * D)
    l1v = jnp.broadcast_to(jnp.asarray(l1_flag, jnp.float32), (L,))
    out = _sc_call(ph, pt, pr, nh, nt, nr, ent2, rel2, transfer, l1v)
    return jnp.sum(out[:, 0])


# speedup vs baseline: 1.1156x; 1.1156x over previous
import functools

import jax
import jax.numpy as jnp
from jax import lax
from jax.experimental import pallas as pl
from jax.experimental.pallas import tpu as pltpu
from jax.experimental.pallas import tpu_sc as plsc

D = 64
B = 4096
L = 16
NQ = D // L
MARGIN = 1.0

_info = plsc.get_sparse_core_info()
NC = _info.num_cores
NS = _info.num_subcores
NW = NC * NS
RPW = B // NW
CH = 8
NCHUNK = RPW // CH


def _lane_iota():
    return lax.iota(jnp.int32, L)


def _lanesum(v):
    iota = _lane_iota()
    for sh in (1, 2, 4, 8):
        v = v + v.at[iota ^ sh].get(mode="promise_in_bounds")
    return v


def _rsqrt(v):
    i = lax.bitcast_convert_type(v, jnp.int32)
    i = jnp.int32(0x5F3759DF) - (i >> 1)
    y = lax.bitcast_convert_type(i, jnp.float32)
    for _ in range(3):
        y = y * (1.5 - 0.5 * v * y * y)
    return y


def _bcast_lane(vec, j):
    return jnp.broadcast_to(vec[j], (L,))


def _row_score(M, H, T, R, rr, mh, mt, mr, l1f):
    def sel(A, f, lo):
        a = A[rr, pl.ds(lo, L)]
        b = A[rr, pl.ds(D + lo, L)]
        return a + f * (b - a)

    z = jnp.zeros((L,), jnp.float32)
    accs = [z] * (2 * NQ)
    for g in range(D // L):
        hch = sel(H, mh, g * L)
        tch = sel(T, mt, g * L)
        for j in range(L):
            bh = _bcast_lane(hch, j)
            bt = _bcast_lane(tch, j)
            dd = g * L + j
            for q in range(NQ):
                m = M[rr, pl.ds(dd * D + q * L, L)]
                accs[q] = accs[q] + bh * m
                accs[NQ + q] = accs[NQ + q] + bt * m
    ah, at = accs[:NQ], accs[NQ:]

    def norm(a):
        ss = a[0] * a[0]
        for q in range(1, NQ):
            ss = ss + a[q] * a[q]
        sv = jnp.maximum(_lanesum(ss), jnp.float32(1e-12))
        y = _rsqrt(sv)
        return [aq * y for aq in a]

    ah = norm(ah)
    at = norm(at)
    sl1 = None
    sl2 = None
    for q in range(NQ):
        rq = sel(R, mr, q * L)
        dq = ah[q] + rq - at[q]
        aq = jnp.abs(dq)
        s2 = dq * dq
        sl1 = aq if sl1 is None else sl1 + aq
        sl2 = s2 if sl2 is None else sl2 + s2
    v1 = _lanesum(sl1)
    v2 = _lanesum(sl2)
    return l1f * v1 + (1.0 - l1f) * v2


def _sc_body(ph, pt, pr, nh, nt, nr, ent2, rel2, tr, l1h, out,
             bph, bpt, bnh, bnt, bpr, bnr, bprs, bnrs,
             oph, opt, opr, onh, ont, onr,
             eph, ept, enh, ent_, erp, ern, mp, mn, l1b, accb, sem):
    c = lax.axis_index("c")
    s = lax.axis_index("s")
    wid = s * NC + c
    base = wid * RPW

    pltpu.sync_copy(ph.at[pl.ds(base, RPW)], bph)
    pltpu.sync_copy(pt.at[pl.ds(base, RPW)], bpt)
    pltpu.sync_copy(nh.at[pl.ds(base, RPW)], bnh)
    pltpu.sync_copy(nt.at[pl.ds(base, RPW)], bnt)
    pltpu.sync_copy(pr.at[pl.ds(base, RPW)], bpr)
    pltpu.sync_copy(nr.at[pl.ds(base, RPW)], bnr)
    pltpu.sync_copy(l1h, l1b)
    l1f = l1b[...]

    def prep(k, _):
        s0 = k * L
        for buf, off in ((bph, oph), (bpt, opt), (bnh, onh), (bnt, ont)):
            v = buf[pl.ds(s0, L)]
            buf[pl.ds(s0, L)] = v >> 1
            off[pl.ds(s0, L)] = v & 1
        for buf, sh, off in ((bpr, bprs, opr), (bnr, bnrs, onr)):
            v = buf[pl.ds(s0, L)]
            sh[pl.ds(s0, L)] = v >> 1
            off[pl.ds(s0, L)] = v & 1
        return 0

    lax.fori_loop(0, RPW // L, prep, 0)

    def chunk(ci, acc):
        c0 = ci * CH
        cps = [
            pltpu.async_copy(ent2.at[bph.at[pl.ds(c0, CH)]], eph, sem),
            pltpu.async_copy(ent2.at[bpt.at[pl.ds(c0, CH)]], ept, sem),
            pltpu.async_copy(ent2.at[bnh.at[pl.ds(c0, CH)]], enh, sem),
            pltpu.async_copy(ent2.at[bnt.at[pl.ds(c0, CH)]], ent_, sem),
            pltpu.async_copy(rel2.at[bprs.at[pl.ds(c0, CH)]], erp, sem),
            pltpu.async_copy(rel2.at[bnrs.at[pl.ds(c0, CH)]], ern, sem),
            pltpu.async_copy(tr.at[bpr.at[pl.ds(c0, CH)]], mp, sem),
            pltpu.async_copy(tr.at[bnr.at[pl.ds(c0, CH)]], mn, sem),
        ]
        for cp in cps:
            cp.wait()

        def row(rr, acc2):
            g = c0 + rr
            wb = (g >> 4) << 4
            ln = jnp.broadcast_to(g & (L - 1), (L,))

            def mask_of(obuf):
                w = obuf[pl.ds(wb, L)]
                return w.at[ln].get(mode="promise_in_bounds").astype(jnp.float32)

            posv = _row_score(mp, eph, ept, erp, rr,
                              mask_of(oph), mask_of(opt), mask_of(opr), l1f)
            negv = _row_score(mn, enh, ent_, ern, rr,
                              mask_of(onh), mask_of(ont), mask_of(onr), l1f)
            return acc2 + jnp.maximum(posv - negv + MARGIN, 0.0)

        return lax.fori_loop(0, CH, row, acc)

    acc = lax.fori_loop(0, NCHUNK, chunk, jnp.zeros((L,), jnp.float32))
    accb[...] = acc
    pltpu.sync_copy(accb, out.at[wid])


_sc_call = functools.partial(
    pl.kernel,
    out_type=jax.ShapeDtypeStruct((NW, L), jnp.float32),
    mesh=plsc.VectorSubcoreMesh(core_axis_name="c", subcore_axis_name="s"),
    scratch_types=(
        [pltpu.VMEM((RPW,), jnp.int32) for _ in range(14)] +
        [pltpu.VMEM((CH, 2 * D), jnp.float32) for _ in range(6)] +
        [pltpu.VMEM((CH, D * D), jnp.float32) for _ in range(2)] +
        [pltpu.VMEM((L,), jnp.float32) for _ in range(2)] +
        [pltpu.SemaphoreType.DMA]
    ),
)(_sc_body)


def kernel(x, ent_emb, rel_emb, transfer, l1_flag):
    ph = x[:, 0]
    pt = x[:, 1]
    pr = x[:, 2]
    nh = x[:, 3]
    nt = x[:, 4]
    nr = x[:, 5]
    ent2 = ent_emb.reshape(ent_emb.shape[0] // 2, 2 * D)
    rel2 = rel_emb.reshape(rel_emb.shape[0] // 2, 2 * D)
    l1v = jnp.broadcast_to(jnp.asarray(l1_flag, jnp.float32), (L,))
    out = _sc_call(ph, pt, pr, nh, nt, nr, ent2, rel2, transfer, l1v)
    return jnp.sum(out[:, 0])
